# Initial kernel scaffold; baseline (speedup 1.0000x reference)
import jax
import jax.numpy as jnp
from jax.experimental import pallas as pl


def kernel(node_feats, edge_attrs, senders, receivers, W0, W1, W2, W3):
    raise NotImplementedError("write your pallas kernel here")


# speedup vs baseline: 2.7279x; 2.7279x over previous
import functools

import numpy as np
import jax
import jax.numpy as jnp
from jax import lax
from jax.experimental import pallas as pl
from jax.experimental.pallas import tpu as pltpu
from jax.experimental.pallas import tpu_sc as plsc

_xs = np.random.RandomState(0).randn(1_000_000)
_silu_np = _xs / (1.0 + np.exp(-_xs))
_SILU_C = float(np.sqrt(np.mean(_silu_np ** 2)))
_INV_SILU_C = 1.0 / _SILU_C

N_NODES = 10000
E = 160000
D = 128
N_EA = 19
CH = 128
NC, NS = 2, 16
NW = NC * NS
E_PAD = 163840
NCHUNK = E_PAD // CH
CPW = NCHUNK // NW
CPT = NCHUNK // NS
ACC_ROWS = 10240
DUMMY_ROW = N_NODES
RPT = N_NODES // NS
ZPT = ACC_ROWS // NS
BE = 1280
BC = BE // CH

_MESH = dict(core_axis_name="c", subcore_axis_name="s",
             num_cores=NC, num_subcores=NS)


def _gather_body(nf, s2d, out, idx_v, rowbuf, sem):
    cid = lax.axis_index("c")
    sid = lax.axis_index("s")
    w = sid * NC + cid
    base = w * CPW
    pltpu.sync_copy(s2d.at[pl.ds(base, CPW)], idx_v)

    def body(j, carry):
        pltpu.async_copy(nf.at[idx_v.at[j]], rowbuf, sem).wait()
        pltpu.sync_copy(rowbuf, out.at[base + j])
        return carry

    lax.fori_loop(0, CPW, body, 0)


@functools.cache
def _gather():
    return pl.kernel(
        _gather_body,
        out_type=jax.ShapeDtypeStruct((NCHUNK, CH, D), jnp.float32),
        mesh=plsc.VectorSubcoreMesh(**_MESH),
        scratch_types=[
            pltpu.VMEM((CPW, CH), jnp.int32),
            pltpu.VMEM((CH, D), jnp.float32),
            pltpu.SemaphoreType.DMA,
        ],
    )


def _tc_body(g_ref, ea_ref, w0, w1, w2, w3, out_ref):
    g = g_ref[...].reshape(BE, D)
    ea = ea_ref[...]
    h = jax.nn.silu((ea[:, :16] @ w0[...]) * 0.25) * _INV_SILU_C
    h = jax.nn.silu((h @ w1[...]) * 0.125) * _INV_SILU_C
    h = jax.nn.silu((h @ w2[...]) * 0.125) * _INV_SILU_C
    mix = (h @ w3[...]) * (0.125 * 0.25)
    ms = mix[:, :D]
    mv = mix[:, D:]
    gv = g * mv
    out = jnp.stack(
        [g * ms, gv * ea[:, 16:17], gv * ea[:, 17:18], gv * ea[:, 18:19]]
    )
    out_ref[...] = out.reshape(4, BC, CH, D)


def _tc_call(gathered, ea_p, W0, W1, W2, W3):
    grid = NCHUNK // BC
    return pl.pallas_call(
        _tc_body,
        grid=(grid,),
        in_specs=[
            pl.BlockSpec((BC, CH, D), lambda i: (i, 0, 0)),
            pl.BlockSpec((BE, N_EA), lambda i: (i, 0)),
            pl.BlockSpec((16, 64), lambda i: (0, 0)),
            pl.BlockSpec((64, 64), lambda i: (0, 0)),
            pl.BlockSpec((64, 64), lambda i: (0, 0)),
            pl.BlockSpec((64, 256), lambda i: (0, 0)),
        ],
        out_specs=pl.BlockSpec((4, BC, CH, D), lambda i: (0, i, 0, 0)),
        out_shape=jax.ShapeDtypeStruct((4, NCHUNK, CH, D), jnp.float32),
    )(gathered, ea_p, W0, W1, W2, W3)


def _scatter_body(msgs, r2d, zeros_hbm, out4, acc, idx_v, mbuf, sem):
    cid = lax.axis_index("c")
    tid = lax.axis_index("s")
    pltpu.sync_copy(r2d.at[pl.ds(tid * CPT, CPT)], idx_v)
    for p in range(2):
        b = cid * 2 + p
        pltpu.sync_copy(zeros_hbm.at[pl.ds(tid * ZPT, ZPT)],
                        acc.at[pl.ds(tid * ZPT, ZPT)])
        plsc.subcore_barrier()

        def body(j, carry):
            gid = tid * CPT + j
            pltpu.async_copy(msgs.at[b, gid], mbuf, sem).wait()
            pltpu.sync_copy(mbuf, acc.at[idx_v.at[j]], add=True)
            return carry

        lax.fori_loop(0, CPT, body, 0)
        plsc.subcore_barrier()
        pltpu.sync_copy(acc.at[pl.ds(tid * ZPT, ZPT)],
                        out4.at[b, pl.ds(tid * ZPT, ZPT)])
        plsc.subcore_barrier()


@functools.cache
def _scatter():
    return pl.kernel(
        _scatter_body,
        out_type=jax.ShapeDtypeStruct((4, ACC_ROWS, D), jnp.float32),
        mesh=plsc.VectorSubcoreMesh(**_MESH),
        scratch_types=[
            pltpu.VMEM_SHARED((ACC_ROWS, D), jnp.float32),
            pltpu.VMEM((CPT, CH), jnp.int32),
            pltpu.VMEM((CH, D), jnp.float32),
            pltpu.SemaphoreType.DMA,
        ],
    )


def kernel(node_feats, edge_attrs, senders, receivers, W0, W1, W2, W3):
    pad = E_PAD - E
    s_p = jnp.concatenate(
        [senders, jnp.zeros((pad,), jnp.int32)]).reshape(NCHUNK, CH)
    r_p = jnp.concatenate(
        [receivers, jnp.full((pad,), DUMMY_ROW, jnp.int32)]).reshape(NCHUNK, CH)
    ea_p = jnp.concatenate(
        [edge_attrs, jnp.zeros((pad, N_EA), jnp.float32)])
    zeros = jnp.zeros((ACC_ROWS, D), jnp.float32)
    gathered = _gather()(node_feats, s_p)
    msgs = _tc_call(gathered, ea_p, W0, W1, W2, W3)
    out4 = _scatter()(msgs, r_p, zeros)[:, :N_NODES]
    out_v = out4[1:4].transpose(1, 2, 0).reshape(N_NODES, 3 * D)
    return jnp.concatenate([out4[0], out_v], axis=1)


# speedup vs baseline: 3.0285x; 1.1102x over previous
import functools

import numpy as np
import jax
import jax.numpy as jnp
from jax import lax
from jax.experimental import pallas as pl
from jax.experimental.pallas import tpu as pltpu
from jax.experimental.pallas import tpu_sc as plsc

_xs = np.random.RandomState(0).randn(1_000_000)
_silu_np = _xs / (1.0 + np.exp(-_xs))
_SILU_C = float(np.sqrt(np.mean(_silu_np ** 2)))
_INV_SILU_C = 1.0 / _SILU_C

N_NODES = 10000
E = 160000
D = 128
N_EA = 19
CH = 128
NC, NS = 2, 16
NW = NC * NS
E_PAD = 163840
NCHUNK = E_PAD // CH
CPW = NCHUNK // NW
CPT = NCHUNK // NS
ACC_ROWS = 10240
DUMMY_ROW = N_NODES
RPT = N_NODES // NS
ZPT = ACC_ROWS // NS
BE = 1280
BC = BE // CH

_MESH = dict(core_axis_name="c", subcore_axis_name="s",
             num_cores=NC, num_subcores=NS)


def _gather_body(nf, s2d, out, idx_v, bufa, bufb, sga, sgb, swa, swb):
    cid = lax.axis_index("c")
    sid = lax.axis_index("s")
    w = sid * NC + cid
    base = w * CPW
    pltpu.sync_copy(s2d.at[pl.ds(base, CPW)], idx_v)

    def body(j2, carry):
        j = j2 * 2
        ga = pltpu.async_copy(nf.at[idx_v.at[j]], bufa, sga)
        gb = pltpu.async_copy(nf.at[idx_v.at[j + 1]], bufb, sgb)
        ga.wait()
        wa = pltpu.async_copy(bufa, out.at[base + j], swa)
        gb.wait()
        wb = pltpu.async_copy(bufb, out.at[base + j + 1], swb)
        wa.wait()
        wb.wait()
        return carry

    lax.fori_loop(0, CPW // 2, body, 0)


@functools.cache
def _gather():
    return pl.kernel(
        _gather_body,
        out_type=jax.ShapeDtypeStruct((NCHUNK, CH, D), jnp.float32),
        mesh=plsc.VectorSubcoreMesh(**_MESH),
        scratch_types=[
            pltpu.VMEM((CPW, CH), jnp.int32),
            pltpu.VMEM((CH, D), jnp.float32),
            pltpu.VMEM((CH, D), jnp.float32),
            pltpu.SemaphoreType.DMA,
            pltpu.SemaphoreType.DMA,
            pltpu.SemaphoreType.DMA,
            pltpu.SemaphoreType.DMA,
        ],
    )


def _tc_body(g_ref, ea_ref, w0, w1, w2, w3, out_ref):
    g = g_ref[...].reshape(BE, D)
    ea = ea_ref[...]
    h = jax.nn.silu((ea[:, :16] @ w0[...]) * 0.25) * _INV_SILU_C
    h = jax.nn.silu((h @ w1[...]) * 0.125) * _INV_SILU_C
    h = jax.nn.silu((h @ w2[...]) * 0.125) * _INV_SILU_C
    mix = (h @ w3[...]) * (0.125 * 0.25)
    ms = mix[:, :D]
    mv = mix[:, D:]
    gv = g * mv
    out = jnp.stack(
        [g * ms, gv * ea[:, 16:17], gv * ea[:, 17:18], gv * ea[:, 18:19]]
    )
    out_ref[...] = out.reshape(4, BC, CH, D)


def _tc_call(gathered, ea_p, W0, W1, W2, W3):
    grid = E // BE
    return pl.pallas_call(
        _tc_body,
        grid=(grid,),
        in_specs=[
            pl.BlockSpec((BC, CH, D), lambda i: (i, 0, 0)),
            pl.BlockSpec((BE, N_EA), lambda i: (i, 0)),
            pl.BlockSpec((16, 64), lambda i: (0, 0)),
            pl.BlockSpec((64, 64), lambda i: (0, 0)),
            pl.BlockSpec((64, 64), lambda i: (0, 0)),
            pl.BlockSpec((64, 256), lambda i: (0, 0)),
        ],
        out_specs=pl.BlockSpec((4, BC, CH, D), lambda i: (0, i, 0, 0)),
        out_shape=jax.ShapeDtypeStruct((4, NCHUNK, CH, D), jnp.float32),
    )(gathered, ea_p, W0, W1, W2, W3)


def _scatter_body(msgs, r2d, zeros_hbm, out4, acc, idx_v, bufa, bufb, sla, slb):
    cid = lax.axis_index("c")
    tid = lax.axis_index("s")
    pltpu.sync_copy(r2d.at[pl.ds(tid * CPT, CPT)], idx_v)
    for p in range(2):
        b = cid * 2 + p
        pltpu.sync_copy(zeros_hbm.at[pl.ds(tid * ZPT, ZPT)],
                        acc.at[pl.ds(tid * ZPT, ZPT)])
        plsc.subcore_barrier()

        def body(j2, carry):
            j = j2 * 2
            gid = tid * CPT + j
            la = pltpu.async_copy(msgs.at[b, gid], bufa, sla)
            lb = pltpu.async_copy(msgs.at[b, gid + 1], bufb, slb)
            la.wait()
            pltpu.sync_copy(bufa, acc.at[idx_v.at[j]], add=True)
            lb.wait()
            pltpu.sync_copy(bufb, acc.at[idx_v.at[j + 1]], add=True)
            return carry

        lax.fori_loop(0, CPT // 2, body, 0)
        plsc.subcore_barrier()
        pltpu.sync_copy(acc.at[pl.ds(tid * ZPT, ZPT)],
                        out4.at[b, pl.ds(tid * ZPT, ZPT)])
        plsc.subcore_barrier()


@functools.cache
def _scatter():
    return pl.kernel(
        _scatter_body,
        out_type=jax.ShapeDtypeStruct((4, ACC_ROWS, D), jnp.float32),
        mesh=plsc.VectorSubcoreMesh(**_MESH),
        scratch_types=[
            pltpu.VMEM_SHARED((ACC_ROWS, D), jnp.float32),
            pltpu.VMEM((CPT, CH), jnp.int32),
            pltpu.VMEM((CH, D), jnp.float32),
            pltpu.VMEM((CH, D), jnp.float32),
            pltpu.SemaphoreType.DMA,
            pltpu.SemaphoreType.DMA,
        ],
    )


def kernel(node_feats, edge_attrs, senders, receivers, W0, W1, W2, W3):
    pad = E_PAD - E
    s_p = jnp.concatenate(
        [senders, jnp.zeros((pad,), jnp.int32)]).reshape(NCHUNK, CH)
    r_p = jnp.concatenate(
        [receivers, jnp.full((pad,), DUMMY_ROW, jnp.int32)]).reshape(NCHUNK, CH)
    zeros = jnp.zeros((ACC_ROWS, D), jnp.float32)
    gathered = _gather()(node_feats, s_p)
    msgs = _tc_call(gathered, edge_attrs, W0, W1, W2, W3)
    out4 = _scatter()(msgs, r_p, zeros)[:, :N_NODES]
    out_v = out4[1:4].transpose(1, 2, 0).reshape(N_NODES, 3 * D)
    return jnp.concatenate([out4[0], out_v], axis=1)


# speedup vs baseline: 3.0318x; 1.0011x over previous
import functools

import numpy as np
import jax
import jax.numpy as jnp
from jax import lax
from jax.experimental import pallas as pl
from jax.experimental.pallas import tpu as pltpu
from jax.experimental.pallas import tpu_sc as plsc

_xs = np.random.RandomState(0).randn(1_000_000)
_silu_np = _xs / (1.0 + np.exp(-_xs))
_SILU_C = float(np.sqrt(np.mean(_silu_np ** 2)))
_INV_SILU_C = 1.0 / _SILU_C

N_NODES = 10000
E = 160000
D = 128
N_EA = 19
CH = 128
NC, NS = 2, 16
NW = NC * NS
E_PAD = 163840
NCHUNK = E_PAD // CH
CPW = NCHUNK // NW
CPT = NCHUNK // NS
ACC_ROWS = 10240
DUMMY_ROW = N_NODES
RPT = N_NODES // NS
ZPT = ACC_ROWS // NS
BE = 1280
BC = BE // CH

_MESH = dict(core_axis_name="c", subcore_axis_name="s",
             num_cores=NC, num_subcores=NS)


def _gather_body(nf, s2d, out, idx_v, bufa, bufb, sga, sgb, swa, swb):
    cid = lax.axis_index("c")
    sid = lax.axis_index("s")
    w = cid * NS + sid
    base = w * CPW
    pltpu.sync_copy(s2d.at[pl.ds(base, CPW)], idx_v)

    def body(j2, carry):
        j = j2 * 2
        ga = pltpu.async_copy(nf.at[idx_v.at[j]], bufa, sga)
        gb = pltpu.async_copy(nf.at[idx_v.at[j + 1]], bufb, sgb)
        ga.wait()
        wa = pltpu.async_copy(bufa, out.at[base + j], swa)
        gb.wait()
        wb = pltpu.async_copy(bufb, out.at[base + j + 1], swb)
        wa.wait()
        wb.wait()
        return carry

    lax.fori_loop(0, CPW // 2, body, 0)


@functools.cache
def _gather():
    return pl.kernel(
        _gather_body,
        out_type=jax.ShapeDtypeStruct((NCHUNK, CH, D), jnp.float32),
        mesh=plsc.VectorSubcoreMesh(**_MESH),
        scratch_types=[
            pltpu.VMEM((CPW, CH), jnp.int32),
            pltpu.VMEM((CH, D), jnp.float32),
            pltpu.VMEM((CH, D), jnp.float32),
            pltpu.SemaphoreType.DMA,
            pltpu.SemaphoreType.DMA,
            pltpu.SemaphoreType.DMA,
            pltpu.SemaphoreType.DMA,
        ],
    )


def _tc_body(g_ref, ea_ref, w0, w1, w2, w3, out_ref):
    g = g_ref[...].reshape(BE, D)
    ea = ea_ref[...]
    h = jax.nn.silu((ea[:, :16] @ w0[...]) * 0.25) * _INV_SILU_C
    h = jax.nn.silu((h @ w1[...]) * 0.125) * _INV_SILU_C
    h = jax.nn.silu((h @ w2[...]) * 0.125) * _INV_SILU_C
    mix = (h @ w3[...]) * (0.125 * 0.25)
    ms = mix[:, :D]
    mv = mix[:, D:]
    gv = g * mv
    out = jnp.stack(
        [g * ms, gv * ea[:, 16:17], gv * ea[:, 17:18], gv * ea[:, 18:19]]
    )
    out_ref[...] = out.reshape(4, BC, CH, D)


def _tc_call(gathered, ea_p, W0, W1, W2, W3):
    grid = E // BE
    return pl.pallas_call(
        _tc_body,
        grid=(grid,),
        in_specs=[
            pl.BlockSpec((BC, CH, D), lambda i: (i, 0, 0)),
            pl.BlockSpec((BE, N_EA), lambda i: (i, 0)),
            pl.BlockSpec((16, 64), lambda i: (0, 0)),
            pl.BlockSpec((64, 64), lambda i: (0, 0)),
            pl.BlockSpec((64, 64), lambda i: (0, 0)),
            pl.BlockSpec((64, 256), lambda i: (0, 0)),
        ],
        out_specs=pl.BlockSpec((4, BC, CH, D), lambda i: (0, i, 0, 0)),
        out_shape=jax.ShapeDtypeStruct((4, NCHUNK, CH, D), jnp.float32),
    )(gathered, ea_p, W0, W1, W2, W3)


def _scatter_body(msgs, r2d, zeros_hbm, out4, acc, idx_v, bufa, bufb, sla, slb):
    cid = lax.axis_index("c")
    tid = lax.axis_index("s")
    pltpu.sync_copy(r2d.at[pl.ds(tid * CPT, CPT)], idx_v)
    for p in range(2):
        b = cid * 2 + p
        pltpu.sync_copy(zeros_hbm.at[pl.ds(tid * ZPT, ZPT)],
                        acc.at[pl.ds(tid * ZPT, ZPT)])
        plsc.subcore_barrier()

        def body(j2, carry):
            j = j2 * 2
            gid = tid * CPT + j
            la = pltpu.async_copy(msgs.at[b, gid], bufa, sla)
            lb = pltpu.async_copy(msgs.at[b, gid + 1], bufb, slb)
            la.wait()
            pltpu.sync_copy(bufa, acc.at[idx_v.at[j]], add=True)
            lb.wait()
            pltpu.sync_copy(bufb, acc.at[idx_v.at[j + 1]], add=True)
            return carry

        lax.fori_loop(0, CPT // 2, body, 0)
        plsc.subcore_barrier()
        pltpu.sync_copy(acc.at[pl.ds(tid * ZPT, ZPT)],
                        out4.at[b, pl.ds(tid * ZPT, ZPT)])
        plsc.subcore_barrier()


@functools.cache
def _scatter():
    return pl.kernel(
        _scatter_body,
        out_type=jax.ShapeDtypeStruct((4, ACC_ROWS, D), jnp.float32),
        mesh=plsc.VectorSubcoreMesh(**_MESH),
        scratch_types=[
            pltpu.VMEM_SHARED((ACC_ROWS, D), jnp.float32),
            pltpu.VMEM((CPT, CH), jnp.int32),
            pltpu.VMEM((CH, D), jnp.float32),
            pltpu.VMEM((CH, D), jnp.float32),
            pltpu.SemaphoreType.DMA,
            pltpu.SemaphoreType.DMA,
        ],
    )


def kernel(node_feats, edge_attrs, senders, receivers, W0, W1, W2, W3):
    pad = E_PAD - E
    s_p = jnp.concatenate(
        [senders, jnp.zeros((pad,), jnp.int32)]).reshape(NCHUNK, CH)
    r_p = jnp.concatenate(
        [receivers, jnp.full((pad,), DUMMY_ROW, jnp.int32)]).reshape(NCHUNK, CH)
    zeros = jnp.zeros((ACC_ROWS, D), jnp.float32)
    gathered = _gather()(node_feats, s_p)
    msgs = _tc_call(gathered, edge_attrs, W0, W1, W2, W3)
    out4 = _scatter()(msgs, r_p, zeros)[:, :N_NODES]
    out_v = out4[1:4].transpose(1, 2, 0).reshape(N_NODES, 3 * D)
    return jnp.concatenate([out4[0], out_v], axis=1)


# speedup vs baseline: 3.0459x; 1.0046x over previous
import functools

import numpy as np
import jax
import jax.numpy as jnp
from jax import lax
from jax.experimental import pallas as pl
from jax.experimental.pallas import tpu as pltpu
from jax.experimental.pallas import tpu_sc as plsc

_xs = np.random.RandomState(0).randn(1_000_000)
_silu_np = _xs / (1.0 + np.exp(-_xs))
_SILU_C = float(np.sqrt(np.mean(_silu_np ** 2)))
_INV_SILU_C = 1.0 / _SILU_C

N_NODES = 10000
E = 160000
D = 128
N_EA = 19
CH = 128
NC, NS = 2, 16
NW = NC * NS
E_PAD = 163840
NCHUNK = E_PAD // CH
CPW = NCHUNK // NW
CPT = NCHUNK // NS
ACC_ROWS = 10240
DUMMY_ROW = N_NODES
RPT = N_NODES // NS
ZPT = ACC_ROWS // NS
BE = 1280
BC = BE // CH

_MESH = dict(core_axis_name="c", subcore_axis_name="s",
             num_cores=NC, num_subcores=NS)


GDEPTH = 4


def _gather_body(nf, s2d, out, idx_v, bufs, gsems, wsems):
    cid = lax.axis_index("c")
    sid = lax.axis_index("s")
    w = cid * NS + sid
    base = w * CPW
    pltpu.sync_copy(s2d.at[pl.ds(base, CPW)], idx_v)

    def body(jr, carry):
        j = jr * GDEPTH
        gs = [pltpu.async_copy(nf.at[idx_v.at[j + k]], bufs[k], gsems[k])
              for k in range(GDEPTH)]
        ws = []
        for k in range(GDEPTH):
            gs[k].wait()
            ws.append(pltpu.async_copy(bufs[k], out.at[base + j + k], wsems[k]))
        for wk in ws:
            wk.wait()
        return carry

    lax.fori_loop(0, CPW // GDEPTH, body, 0)


@functools.cache
def _gather():
    return pl.kernel(
        _gather_body,
        out_type=jax.ShapeDtypeStruct((NCHUNK, CH, D), jnp.float32),
        mesh=plsc.VectorSubcoreMesh(**_MESH),
        scratch_types=[
            pltpu.VMEM((CPW, CH), jnp.int32),
            [pltpu.VMEM((CH, D), jnp.float32) for _ in range(GDEPTH)],
            [pltpu.SemaphoreType.DMA for _ in range(GDEPTH)],
            [pltpu.SemaphoreType.DMA for _ in range(GDEPTH)],
        ],
    )


def _tc_body(g_ref, ea_ref, w0, w1, w2, w3, out_ref):
    g = g_ref[...].reshape(BE, D)
    ea = ea_ref[...]
    h = jax.nn.silu((ea[:, :16] @ w0[...]) * 0.25) * _INV_SILU_C
    h = jax.nn.silu((h @ w1[...]) * 0.125) * _INV_SILU_C
    h = jax.nn.silu((h @ w2[...]) * 0.125) * _INV_SILU_C
    mix = (h @ w3[...]) * (0.125 * 0.25)
    ms = mix[:, :D]
    mv = mix[:, D:]
    gv = g * mv
    out = jnp.stack(
        [g * ms, gv * ea[:, 16:17], gv * ea[:, 17:18], gv * ea[:, 18:19]]
    )
    out_ref[...] = out.reshape(4, BC, CH, D)


def _tc_call(gathered, ea_p, W0, W1, W2, W3):
    grid = E // BE
    return pl.pallas_call(
        _tc_body,
        grid=(grid,),
        in_specs=[
            pl.BlockSpec((BC, CH, D), lambda i: (i, 0, 0)),
            pl.BlockSpec((BE, N_EA), lambda i: (i, 0)),
            pl.BlockSpec((16, 64), lambda i: (0, 0)),
            pl.BlockSpec((64, 64), lambda i: (0, 0)),
            pl.BlockSpec((64, 64), lambda i: (0, 0)),
            pl.BlockSpec((64, 256), lambda i: (0, 0)),
        ],
        out_specs=pl.BlockSpec((4, BC, CH, D), lambda i: (0, i, 0, 0)),
        out_shape=jax.ShapeDtypeStruct((4, NCHUNK, CH, D), jnp.float32),
    )(gathered, ea_p, W0, W1, W2, W3)


def _scatter_body(msgs, r2d, zeros_hbm, out4, acc, idx_v, bufa, bufb, sla, slb):
    cid = lax.axis_index("c")
    tid = lax.axis_index("s")
    pltpu.sync_copy(r2d.at[pl.ds(tid * CPT, CPT)], idx_v)
    for p in range(2):
        b = cid * 2 + p
        pltpu.sync_copy(zeros_hbm.at[pl.ds(tid * ZPT, ZPT)],
                        acc.at[pl.ds(tid * ZPT, ZPT)])
        plsc.subcore_barrier()

        def body(j2, carry):
            j = j2 * 2
            gid = tid * CPT + j
            la = pltpu.async_copy(msgs.at[b, gid], bufa, sla)
            lb = pltpu.async_copy(msgs.at[b, gid + 1], bufb, slb)
            la.wait()
            pltpu.sync_copy(bufa, acc.at[idx_v.at[j]], add=True)
            lb.wait()
            pltpu.sync_copy(bufb, acc.at[idx_v.at[j + 1]], add=True)
            return carry

        lax.fori_loop(0, CPT // 2, body, 0)
        plsc.subcore_barrier()
        pltpu.sync_copy(acc.at[pl.ds(tid * ZPT, ZPT)],
                        out4.at[b, pl.ds(tid * ZPT, ZPT)])
        plsc.subcore_barrier()


@functools.cache
def _scatter():
    return pl.kernel(
        _scatter_body,
        out_type=jax.ShapeDtypeStruct((4, ACC_ROWS, D), jnp.float32),
        mesh=plsc.VectorSubcoreMesh(**_MESH),
        scratch_types=[
            pltpu.VMEM_SHARED((ACC_ROWS, D), jnp.float32),
            pltpu.VMEM((CPT, CH), jnp.int32),
            pltpu.VMEM((CH, D), jnp.float32),
            pltpu.VMEM((CH, D), jnp.float32),
            pltpu.SemaphoreType.DMA,
            pltpu.SemaphoreType.DMA,
        ],
    )


def kernel(node_feats, edge_attrs, senders, receivers, W0, W1, W2, W3):
    pad = E_PAD - E
    s_p = jnp.concatenate(
        [senders, jnp.zeros((pad,), jnp.int32)]).reshape(NCHUNK, CH)
    r_p = jnp.concatenate(
        [receivers, jnp.full((pad,), DUMMY_ROW, jnp.int32)]).reshape(NCHUNK, CH)
    zeros = jnp.zeros((ACC_ROWS, D), jnp.float32)
    gathered = _gather()(node_feats, s_p)
    msgs = _tc_call(gathered, edge_attrs, W0, W1, W2, W3)
    out4 = _scatter()(msgs, r_p, zeros)[:, :N_NODES]
    out_v = out4[1:4].transpose(1, 2, 0).reshape(N_NODES, 3 * D)
    return jnp.concatenate([out4[0], out_v], axis=1)


# speedup vs baseline: 4.1292x; 1.3557x over previous
import functools

import numpy as np
import jax
import jax.numpy as jnp
from jax import lax
from jax.experimental import pallas as pl
from jax.experimental.pallas import tpu as pltpu
from jax.experimental.pallas import tpu_sc as plsc

_xs = np.random.RandomState(0).randn(1_000_000)
_silu_np = _xs / (1.0 + np.exp(-_xs))
_SILU_C = float(np.sqrt(np.mean(_silu_np ** 2)))
_INV_SILU_C = 1.0 / _SILU_C

N_NODES = 10000
E = 160000
D = 128
N_EA = 19
CH = 128
NC, NS = 2, 16
NW = NC * NS
E_PAD = 163840
NCHUNK = E_PAD // CH
CPW = NCHUNK // NW
CPT = NCHUNK // NS
ACC_ROWS = 10240
DUMMY_ROW = N_NODES
RPT = N_NODES // NS
ZPT = ACC_ROWS // NS
BE = 1280
BC = BE // CH

_MESH = dict(core_axis_name="c", subcore_axis_name="s",
             num_cores=NC, num_subcores=NS)


GDEPTH = 2


def _gather_body(nf, s2d, out, spt, idx_v, bufs, gsems, wsems):
    cid = lax.axis_index("c")
    sid = lax.axis_index("s")
    w = cid * NS + sid
    base = w * CPW
    @pl.when(sid < NS - 1)
    def _():
        pltpu.sync_copy(nf.at[pl.ds(sid * 640, 640)],
                        spt.at[pl.ds(sid * 640, 640)])

    @pl.when(sid == NS - 1)
    def _():
        pltpu.sync_copy(nf.at[pl.ds((NS - 1) * 640, N_NODES - (NS - 1) * 640)],
                        spt.at[pl.ds((NS - 1) * 640, N_NODES - (NS - 1) * 640)])

    pltpu.sync_copy(s2d.at[pl.ds(base, CPW)], idx_v)
    plsc.subcore_barrier()

    def body(jr, carry):
        j = jr * GDEPTH
        gs = [pltpu.async_copy(spt.at[idx_v.at[j + k]], bufs[k], gsems[k])
              for k in range(GDEPTH)]
        ws = []
        for k in range(GDEPTH):
            gs[k].wait()
            ws.append(pltpu.async_copy(bufs[k], out.at[base + j + k], wsems[k]))
        for wk in ws:
            wk.wait()
        return carry

    lax.fori_loop(0, CPW // GDEPTH, body, 0)


@functools.cache
def _gather():
    return pl.kernel(
        _gather_body,
        out_type=jax.ShapeDtypeStruct((NCHUNK, CH, D), jnp.float32),
        mesh=plsc.VectorSubcoreMesh(**_MESH),
        scratch_types=[
            pltpu.VMEM_SHARED((N_NODES, D), jnp.float32),
            pltpu.VMEM((CPW, CH), jnp.int32),
            [pltpu.VMEM((CH, D), jnp.float32) for _ in range(GDEPTH)],
            [pltpu.SemaphoreType.DMA for _ in range(GDEPTH)],
            [pltpu.SemaphoreType.DMA for _ in range(GDEPTH)],
        ],
    )


def _tc_body(g_ref, ea_ref, w0, w1, w2, w3, out_ref):
    g = g_ref[...].reshape(BE, D)
    ea = ea_ref[...]
    h = jax.nn.silu((ea[:, :16] @ w0[...]) * 0.25) * _INV_SILU_C
    h = jax.nn.silu((h @ w1[...]) * 0.125) * _INV_SILU_C
    h = jax.nn.silu((h @ w2[...]) * 0.125) * _INV_SILU_C
    mix = (h @ w3[...]) * (0.125 * 0.25)
    ms = mix[:, :D]
    mv = mix[:, D:]
    gv = g * mv
    out = jnp.stack(
        [g * ms, gv * ea[:, 16:17], gv * ea[:, 17:18], gv * ea[:, 18:19]]
    )
    out_ref[...] = out.reshape(4, BC, CH, D)


def _tc_call(gathered, ea_p, W0, W1, W2, W3):
    grid = E // BE
    return pl.pallas_call(
        _tc_body,
        grid=(grid,),
        in_specs=[
            pl.BlockSpec((BC, CH, D), lambda i: (i, 0, 0)),
            pl.BlockSpec((BE, N_EA), lambda i: (i, 0)),
            pl.BlockSpec((16, 64), lambda i: (0, 0)),
            pl.BlockSpec((64, 64), lambda i: (0, 0)),
            pl.BlockSpec((64, 64), lambda i: (0, 0)),
            pl.BlockSpec((64, 256), lambda i: (0, 0)),
        ],
        out_specs=pl.BlockSpec((4, BC, CH, D), lambda i: (0, i, 0, 0)),
        out_shape=jax.ShapeDtypeStruct((4, NCHUNK, CH, D), jnp.float32),
    )(gathered, ea_p, W0, W1, W2, W3)


def _scatter_body(msgs, r2d, zeros_hbm, out4, acc, idx_v, bufa, bufb, sla, slb):
    cid = lax.axis_index("c")
    tid = lax.axis_index("s")
    pltpu.sync_copy(r2d.at[pl.ds(tid * CPT, CPT)], idx_v)
    for p in range(2):
        b = cid * 2 + p
        pltpu.sync_copy(zeros_hbm.at[pl.ds(tid * ZPT, ZPT)],
                        acc.at[pl.ds(tid * ZPT, ZPT)])
        plsc.subcore_barrier()

        def body(j2, carry):
            j = j2 * 2
            gid = tid * CPT + j
            la = pltpu.async_copy(msgs.at[b, gid], bufa, sla)
            lb = pltpu.async_copy(msgs.at[b, gid + 1], bufb, slb)
            la.wait()
            pltpu.sync_copy(bufa, acc.at[idx_v.at[j]], add=True)
            lb.wait()
            pltpu.sync_copy(bufb, acc.at[idx_v.at[j + 1]], add=True)
            return carry

        lax.fori_loop(0, CPT // 2, body, 0)
        plsc.subcore_barrier()
        pltpu.sync_copy(acc.at[pl.ds(tid * ZPT, ZPT)],
                        out4.at[b, pl.ds(tid * ZPT, ZPT)])
        plsc.subcore_barrier()


@functools.cache
def _scatter():
    return pl.kernel(
        _scatter_body,
        out_type=jax.ShapeDtypeStruct((4, ACC_ROWS, D), jnp.float32),
        mesh=plsc.VectorSubcoreMesh(**_MESH),
        scratch_types=[
            pltpu.VMEM_SHARED((ACC_ROWS, D), jnp.float32),
            pltpu.VMEM((CPT, CH), jnp.int32),
            pltpu.VMEM((CH, D), jnp.float32),
            pltpu.VMEM((CH, D), jnp.float32),
            pltpu.SemaphoreType.DMA,
            pltpu.SemaphoreType.DMA,
        ],
    )


def kernel(node_feats, edge_attrs, senders, receivers, W0, W1, W2, W3):
    pad = E_PAD - E
    s_p = jnp.concatenate(
        [senders, jnp.zeros((pad,), jnp.int32)]).reshape(NCHUNK, CH)
    r_p = jnp.concatenate(
        [receivers, jnp.full((pad,), DUMMY_ROW, jnp.int32)]).reshape(NCHUNK, CH)
    zeros = jnp.zeros((ACC_ROWS, D), jnp.float32)
    gathered = _gather()(node_feats, s_p)
    msgs = _tc_call(gathered, edge_attrs, W0, W1, W2, W3)
    out4 = _scatter()(msgs, r_p, zeros)[:, :N_NODES]
    out_v = out4[1:4].transpose(1, 2, 0).reshape(N_NODES, 3 * D)
    return jnp.concatenate([out4[0], out_v], axis=1)


# speedup vs baseline: 4.5761x; 1.1082x over previous
import functools

import numpy as np
import jax
import jax.numpy as jnp
from jax import lax
from jax.experimental import pallas as pl
from jax.experimental.pallas import tpu as pltpu
from jax.experimental.pallas import tpu_sc as plsc

_xs = np.random.RandomState(0).randn(1_000_000)
_silu_np = _xs / (1.0 + np.exp(-_xs))
_SILU_C = float(np.sqrt(np.mean(_silu_np ** 2)))
_INV_SILU_C = 1.0 / _SILU_C

N_NODES = 10000
E = 160000
D = 128
N_EA = 19
CH = 128
NC, NS = 2, 16
NW = NC * NS
E_PAD = 163840
NCHUNK = E_PAD // CH
CPW = NCHUNK // NW
CPT = NCHUNK // NS
ACC_ROWS = 10240
DUMMY_ROW = N_NODES
RPT = N_NODES // NS
ZPT = ACC_ROWS // NS
BE = 1280
BC = BE // CH

_MESH = dict(core_axis_name="c", subcore_axis_name="s",
             num_cores=NC, num_subcores=NS)


GDEPTH = 2


def _gather_body(nf, s2d, out, spt, idx_v, bufs, gsems, wsems):
    cid = lax.axis_index("c")
    sid = lax.axis_index("s")
    w = cid * NS + sid
    base = w * CPW
    @pl.when(sid < NS - 1)
    def _():
        pltpu.sync_copy(nf.at[pl.ds(sid * 640, 640)],
                        spt.at[pl.ds(sid * 640, 640)])

    @pl.when(sid == NS - 1)
    def _():
        pltpu.sync_copy(nf.at[pl.ds((NS - 1) * 640, N_NODES - (NS - 1) * 640)],
                        spt.at[pl.ds((NS - 1) * 640, N_NODES - (NS - 1) * 640)])

    pltpu.sync_copy(s2d.at[pl.ds(base, CPW)], idx_v)
    plsc.subcore_barrier()

    def body(jr, carry):
        j = jr * GDEPTH
        gs = [pltpu.async_copy(spt.at[idx_v.at[j + k]], bufs[k], gsems[k])
              for k in range(GDEPTH)]
        ws = []
        for k in range(GDEPTH):
            gs[k].wait()
            ws.append(pltpu.async_copy(bufs[k], out.at[base + j + k], wsems[k]))
        for wk in ws:
            wk.wait()
        return carry

    lax.fori_loop(0, CPW // GDEPTH, body, 0)


@functools.cache
def _gather():
    return pl.kernel(
        _gather_body,
        out_type=jax.ShapeDtypeStruct((NCHUNK, CH, D), jnp.float32),
        mesh=plsc.VectorSubcoreMesh(**_MESH),
        scratch_types=[
            pltpu.VMEM_SHARED((N_NODES, D), jnp.float32),
            pltpu.VMEM((CPW, CH), jnp.int32),
            [pltpu.VMEM((CH, D), jnp.float32) for _ in range(GDEPTH)],
            [pltpu.SemaphoreType.DMA for _ in range(GDEPTH)],
            [pltpu.SemaphoreType.DMA for _ in range(GDEPTH)],
        ],
    )


def _tc_body(g_ref, ea_ref, w0, w1, w2, w3, out_ref):
    g = g_ref[...].reshape(BE, D)
    ea = ea_ref[...]
    h = jax.nn.silu((ea[:, :16] @ w0[...]) * 0.25) * _INV_SILU_C
    h = jax.nn.silu((h @ w1[...]) * 0.125) * _INV_SILU_C
    h = jax.nn.silu((h @ w2[...]) * 0.125) * _INV_SILU_C
    mix = (h @ w3[...]) * (0.125 * 0.25)
    ms = mix[:, :D]
    mv = mix[:, D:]
    gv = g * mv
    out = jnp.stack(
        [g * ms, gv * ea[:, 16:17], gv * ea[:, 17:18], gv * ea[:, 18:19]]
    )
    out_ref[...] = out.reshape(4, BC, CH, D)


def _tc_call(gathered, ea_p, W0, W1, W2, W3):
    grid = E // BE
    return pl.pallas_call(
        _tc_body,
        grid=(grid,),
        in_specs=[
            pl.BlockSpec((BC, CH, D), lambda i: (i, 0, 0)),
            pl.BlockSpec((BE, N_EA), lambda i: (i, 0)),
            pl.BlockSpec((16, 64), lambda i: (0, 0)),
            pl.BlockSpec((64, 64), lambda i: (0, 0)),
            pl.BlockSpec((64, 64), lambda i: (0, 0)),
            pl.BlockSpec((64, 256), lambda i: (0, 0)),
        ],
        out_specs=pl.BlockSpec((4, BC, CH, D), lambda i: (0, i, 0, 0)),
        out_shape=jax.ShapeDtypeStruct((4, NCHUNK, CH, D), jnp.float32),
    )(gathered, ea_p, W0, W1, W2, W3)


def _scatter_body(msgs, r2d, zeros_hbm, out4, acc, idx_v, bufa, bufb, sla, slb):
    cid = lax.axis_index("c")
    tid = lax.axis_index("s")
    pltpu.sync_copy(r2d.at[pl.ds(tid * CPT, CPT)], idx_v)
    for p in range(2):
        b = cid * 2 + p
        pltpu.sync_copy(zeros_hbm.at[pl.ds(tid * ZPT, ZPT)],
                        acc.at[pl.ds(tid * ZPT, ZPT)])
        plsc.subcore_barrier()

        def body(j2, carry):
            j = j2 * 2
            gid = tid * CPT + j
            la = pltpu.async_copy(msgs.at[b, gid], bufa, sla)
            lb = pltpu.async_copy(msgs.at[b, gid + 1], bufb, slb)
            la.wait()
            pltpu.sync_copy(bufa, acc.at[idx_v.at[j]], add=True)
            lb.wait()
            pltpu.sync_copy(bufb, acc.at[idx_v.at[j + 1]], add=True)
            return carry

        lax.fori_loop(0, CPT // 2, body, 0)
        plsc.subcore_barrier()
        pltpu.sync_copy(acc.at[pl.ds(tid * ZPT, ZPT)],
                        out4.at[b, pl.ds(tid * ZPT, ZPT)])
        plsc.subcore_barrier()


@functools.cache
def _scatter():
    return pl.kernel(
        _scatter_body,
        out_type=jax.ShapeDtypeStruct((4, ACC_ROWS, D), jnp.float32),
        mesh=plsc.VectorSubcoreMesh(**_MESH),
        scratch_types=[
            pltpu.VMEM_SHARED((ACC_ROWS, D), jnp.float32),
            pltpu.VMEM((CPT, CH), jnp.int32),
            pltpu.VMEM((CH, D), jnp.float32),
            pltpu.VMEM((CH, D), jnp.float32),
            pltpu.SemaphoreType.DMA,
            pltpu.SemaphoreType.DMA,
        ],
    )


def _ep_body(o4_ref, p_ref, out_ref):
    s = o4_ref[0]
    v = jnp.concatenate([o4_ref[1], o4_ref[2], o4_ref[3]], axis=1)
    out_ref[...] = jnp.concatenate([s, v @ p_ref[...]], axis=1)


def _ep_call(out4, perm):
    RB = 2000
    return pl.pallas_call(
        _ep_body,
        grid=(N_NODES // RB,),
        in_specs=[
            pl.BlockSpec((4, RB, D), lambda i: (0, i, 0)),
            pl.BlockSpec((3 * D, 3 * D), lambda i: (0, 0)),
        ],
        out_specs=pl.BlockSpec((RB, 4 * D), lambda i: (i, 0)),
        out_shape=jax.ShapeDtypeStruct((N_NODES, 4 * D), jnp.float32),
    )(out4, perm)


_PSRC = np.arange(3 * D)
_PERM_NP = np.zeros((3 * D, 3 * D), dtype=np.float32)
_PERM_NP[(_PSRC % 3) * D + _PSRC // 3, _PSRC] = 1.0


def kernel(node_feats, edge_attrs, senders, receivers, W0, W1, W2, W3):
    pad = E_PAD - E
    s_p = jnp.concatenate(
        [senders, jnp.zeros((pad,), jnp.int32)]).reshape(NCHUNK, CH)
    r_p = jnp.concatenate(
        [receivers, jnp.full((pad,), DUMMY_ROW, jnp.int32)]).reshape(NCHUNK, CH)
    zeros = jnp.zeros((ACC_ROWS, D), jnp.float32)
    gathered = _gather()(node_feats, s_p)
    msgs = _tc_call(gathered, edge_attrs, W0, W1, W2, W3)
    out4 = _scatter()(msgs, r_p, zeros)
    return _ep_call(out4, jnp.asarray(_PERM_NP))
